# baseline (speedup 1.0000x reference)
import jax
import jax.numpy as jnp
from jax.experimental import pallas as pl

_E = 16
_C = 768
_HW = 576
_BB = 8


def _gate_kernel(x_ref, wt_ref, b_ref, out_ref):
    for i in range(_BB):
        xb = x_ref[i]
        s = jnp.sum(xb, axis=1, keepdims=True)
        prod = s * wt_ref[...]
        logits = (jnp.sum(prod, axis=0, keepdims=True) * (1.0 / _HW)
                  + b_ref[...])
        iota = jax.lax.broadcasted_iota(jnp.int32, (1, _E), 1)
        m = jnp.max(logits, axis=1, keepdims=True)
        first = jnp.min(jnp.where(logits == m, iota, _E),
                        axis=1, keepdims=True)
        out_ref[i] = (iota == first).astype(jnp.float32)


def kernel(x, W, b):
    B = x.shape[0]
    x3 = x.reshape(B, _C, _HW)
    wt = W.T
    b2 = b.reshape(1, _E)
    out = pl.pallas_call(
        _gate_kernel,
        grid=(B // _BB,),
        in_specs=[
            pl.BlockSpec((_BB, _C, _HW), lambda i: (i, 0, 0)),
            pl.BlockSpec((_C, _E), lambda i: (0, 0)),
            pl.BlockSpec((1, _E), lambda i: (0, 0)),
        ],
        out_specs=pl.BlockSpec((_BB, 1, _E), lambda i: (i, 0, 0)),
        out_shape=jax.ShapeDtypeStruct((B, 1, _E), jnp.float32),
    )(x3, wt, b2)
    return out.reshape(B, _E)


# speedup vs baseline: 1.0228x; 1.0228x over previous
import jax
import jax.numpy as jnp
from jax.experimental import pallas as pl
from jax.experimental.pallas import tpu as pltpu

_E = 16
_C = 768
_HW = 576
_BB = 4


def _gate_kernel(x_ref, wt_ref, b_ref, out_ref):
    for i in range(_BB):
        xb = x_ref[i]
        s = jnp.sum(xb, axis=1, keepdims=True)
        prod = s * wt_ref[...]
        logits = (jnp.sum(prod, axis=0, keepdims=True) * (1.0 / _HW)
                  + b_ref[...])
        iota = jax.lax.broadcasted_iota(jnp.int32, (1, _E), 1)
        m = jnp.max(logits, axis=1, keepdims=True)
        first = jnp.min(jnp.where(logits == m, iota, _E),
                        axis=1, keepdims=True)
        out_ref[i] = (iota == first).astype(jnp.float32)


def kernel(x, W, b):
    B = x.shape[0]
    x3 = x.reshape(B, _C, _HW)
    wt = W.T
    b2 = b.reshape(1, _E)
    out = pl.pallas_call(
        _gate_kernel,
        grid=(B // _BB,),
        in_specs=[
            pl.BlockSpec((_BB, _C, _HW), lambda i: (i, 0, 0)),
            pl.BlockSpec((_C, _E), lambda i: (0, 0)),
            pl.BlockSpec((1, _E), lambda i: (0, 0)),
        ],
        out_specs=pl.BlockSpec((_BB, 1, _E), lambda i: (i, 0, 0)),
        out_shape=jax.ShapeDtypeStruct((B, 1, _E), jnp.float32),
        compiler_params=pltpu.CompilerParams(
            allow_input_fusion=[True, False, False]),
    )(x3, wt, b2)
    return out.reshape(B, _E)
